# baseline (speedup 1.0000x reference)
import functools

import jax
import jax.numpy as jnp
from jax import lax
from jax.experimental import pallas as pl
from jax.experimental.pallas import tpu as pltpu
from jax.experimental.pallas import tpu_sc as plsc

V = 1000
H = 4
D = 1000
DP = 1024

_NC = 2
_NS = 16
_NW = _NC * _NS
_LANES = 16


def _table_kernel(emb_ref, w_ref, b_ref, t_ref):
    t_ref[...] = (
        jnp.dot(emb_ref[...], w_ref[...], preferred_element_type=jnp.float32)
        + b_ref[...]
    )


def _make_gather(Bt, Lt):
    items_per_w = Bt // _NW
    mesh = plsc.VectorSubcoreMesh(core_axis_name="c", subcore_axis_name="s")

    def _relayout(g2d, s3d, buf):
        for r in range(Lt):
            for c in range(D // _LANES):
                s3d[buf, r, pl.ds(c * _LANES, _LANES)] = g2d[
                    buf, r, pl.ds(c * _LANES, _LANES)
                ]
            s3d[buf, r, pl.ds(D - _LANES, _LANES)] = g2d[
                buf, r, pl.ds(D - _LANES, _LANES)
            ]

    LtP = 24

    def _body(table_hbm, idx_hbm, out_hbm, idx_v, g2d, s3d, gsem0, gsem1):
        cid = lax.axis_index("c")
        sid = lax.axis_index("s")
        wid = sid * _NC + cid
        item0 = wid * items_per_w

        pltpu.sync_copy(idx_hbm.at[pl.ds(item0, items_per_w)], idx_v)

        def start_gather(k, buf, sem):
            pltpu.async_copy(
                table_hbm.at[idx_v.at[k]], g2d.at[buf], sem
            )

        def wait_gather(buf, sem):
            pltpu.make_async_copy(
                table_hbm.at[pl.ds(0, LtP)], g2d.at[buf], sem
            ).wait()

        def emit(k, buf):
            _relayout(g2d, s3d, buf)
            pltpu.sync_copy(s3d.at[buf], out_hbm.at[item0 + k])

        start_gather(0, 0, gsem0)

        def body(i, carry):
            k0 = 2 * i
            start_gather(k0 + 1, 1, gsem1)
            wait_gather(0, gsem0)
            emit(k0, 0)
            start_gather(jnp.minimum(k0 + 2, items_per_w - 1), 0, gsem0)
            wait_gather(1, gsem1)
            emit(k0 + 1, 1)
            return carry

        lax.fori_loop(0, items_per_w // 2, body, 0)
        wait_gather(0, gsem0)

    @functools.partial(
        pl.kernel,
        mesh=mesh,
        compiler_params=pltpu.CompilerParams(use_tc_tiling_on_sc=True),
        out_type=jax.ShapeDtypeStruct((Bt, Lt, D), jnp.float32),
        scratch_types=[
            pltpu.VMEM((Bt // _NW, 24), jnp.int32),
            pltpu.VMEM((2, 24, DP), jnp.float32),
            pltpu.VMEM((2, Lt, D), jnp.float32),
            pltpu.SemaphoreType.DMA,
            pltpu.SemaphoreType.DMA,
        ],
    )
    def gather(table_hbm, idx_hbm, out_hbm, idx_v, g2d, s3d, gsem0, gsem1):
        _body(table_hbm, idx_hbm, out_hbm, idx_v, g2d, s3d, gsem0, gsem1)

    return gather


def kernel(input_ids, emb, W, b):
    Bt, Lt = input_ids.shape
    w_pad = jnp.pad(W, ((0, 0), (0, DP - D)))
    b_pad = jnp.pad(b, (0, DP - D)).reshape(1, DP)
    table = pl.pallas_call(
        _table_kernel,
        out_shape=jax.ShapeDtypeStruct((V, DP), jnp.float32),
    )(emb, w_pad, b_pad)

    ids = input_ids.astype(jnp.int32)
    ids24 = jnp.pad(ids, ((0, 0), (0, 24 - Lt)), mode="edge")
    return _make_gather(Bt, Lt)(table, ids24)
